# Initial kernel scaffold; baseline (speedup 1.0000x reference)
import jax
import jax.numpy as jnp
from jax.experimental import pallas as pl


def kernel(x, params):
    raise NotImplementedError("write your pallas kernel here")


# speedup vs baseline: 1.6852x; 1.6852x over previous
import jax
import jax.numpy as jnp
from jax.experimental import pallas as pl
from jax.experimental.pallas import tpu as pltpu

D, H, W = 5, 48, 48
N = D * H * W
PLANE = H * W
P0 = 2360
NP = N + 2 * P0
K_TOP = max(1, int(0.5 * N))
NR = N // 128
BK = 256
NB = N // BK
BR = BK // 128
EPS = 1e-3

_f32 = jnp.float32
_bf16 = jnp.bfloat16


def _row_masks(b, dy, dx):
    r = b * BK + jax.lax.broadcasted_iota(jnp.int32, (BK, 1), 0)
    rem = r % PLANE
    m = None
    if dy == -1:
        m = rem // W >= 1
    elif dy == 1:
        m = rem // W <= H - 2
    if dx == -1:
        mw = rem % W >= 1
        m = mw if m is None else m & mw
    elif dx == 1:
        mw = rem % W <= W - 2
        m = mw if m is None else m & mw
    return None if m is None else m.astype(_f32)


def _conv_block(pad_ref, w_ref, wrow0, cin, b):
    pieces = []
    for dz in (-1, 0, 1):
        for dy in (-1, 0, 1):
            for dx in (-1, 0, 1):
                s = dz * PLANE + dy * W + dx
                sl = pad_ref[pl.ds(P0 + s + b * BK, BK), pl.ds(0, cin)]
                m = _row_masks(b, dy, dx)
                if m is not None:
                    sl = sl * m
                pieces.append(sl.astype(_bf16))
    g = jnp.concatenate(pieces, axis=1)
    wg = w_ref[pl.ds(wrow0, 27 * cin), :]
    return jnp.dot(g, wg, preferred_element_type=_f32)


def _conv_pass(pad_ref, w_ref, cin, cout, dst, col, wrow0=0):
    def blk(b, s_sum):
        acc = _conv_block(pad_ref, w_ref, wrow0, cin, b)
        dst[pl.ds(b * BK, BK), pl.ds(col, cout)] = acc
        return s_sum + jnp.sum(acc, axis=0, keepdims=True)

    s_sum = jax.lax.fori_loop(0, NB, blk, jnp.zeros((1, cout), _f32))
    m = s_sum / N

    def blk2(b, s_sq):
        d = dst[pl.ds(b * BK, BK), pl.ds(col, cout)] - m
        return s_sq + jnp.sum(d * d, axis=0, keepdims=True)

    s_sq = jax.lax.fori_loop(0, NB, blk2, jnp.zeros((1, cout), _f32))
    return m, s_sq / N


def _norm(xv, m, v, g, bta):
    return (xv - m) / jnp.sqrt(v + EPS) * g + bta


def _conv_bn(pad_ref, w_ref, g_ref, b_ref, cin, cout, dst, col, relu):
    m, v = _conv_pass(pad_ref, w_ref, cin, cout, dst, col)
    g = g_ref[:, pl.ds(0, cout)]
    bta = b_ref[:, pl.ds(0, cout)]

    def norm(b, _):
        blkv = _norm(dst[pl.ds(b * BK, BK), pl.ds(col, cout)], m, v, g, bta)
        if relu:
            blkv = jnp.maximum(blkv, 0.0)
        dst[pl.ds(b * BK, BK), pl.ds(col, cout)] = blkv
        return 0

    jax.lax.fori_loop(0, NB, norm, 0)


def _topk_mask(s2r):
    r1 = jax.lax.broadcasted_iota(jnp.int32, (128, 128), 0)
    c1 = jax.lax.broadcasted_iota(jnp.int32, (128, 128), 1)
    tri128 = (r1 <= c1).astype(_f32)
    r9 = jax.lax.broadcasted_iota(jnp.int32, (NR, NR), 0)
    c9 = jax.lax.broadcasted_iota(jnp.int32, (NR, NR), 1)
    tl = (c9 < r9).astype(_f32)
    imp2 = s2r[:, :]
    bits = jax.lax.bitcast_convert_type(imp2, jnp.int32)

    def it(i, carry):
        lo, hi = carry
        mid = (lo + hi) // 2
        cnt = jnp.sum((bits >= mid).astype(jnp.int32))
        ok = cnt >= K_TOP
        return (jnp.where(ok, mid, lo), jnp.where(ok, hi, mid))

    lo, _ = jax.lax.fori_loop(0, 31, it,
                              (jnp.int32(0), jnp.int32(0x40000000)))
    cgt = jnp.sum((bits > lo).astype(jnp.int32))
    need = (K_TOP - cgt).astype(_f32)
    eq = (bits == lo).astype(_f32)
    incl = jnp.dot(eq, tri128, preferred_element_type=_f32)
    rowsum = jnp.sum(eq, axis=1, keepdims=True)
    off = jnp.dot(tl, rowsum, preferred_element_type=_f32)
    keep_eq = jnp.where(incl + off <= need, eq, 0.0)
    maskf = (bits > lo).astype(_f32) + keep_eq
    s2r[:, :] = imp2 * maskf


def _focal_stage(src, scol, wi_ref, bi_ref, pad_ref, cin, s2r):
    def logits_blk(b, _):
        xb = src[pl.ds(b * BK, BK), pl.ds(scol, cin)]
        wv = wi_ref[:, pl.ds(0, cin)]
        xbb = xb.astype(_bf16).astype(_f32)
        wvb = wv.astype(_bf16).astype(_f32)
        lb = jnp.sum(xbb * wvb, axis=1, keepdims=True) + bi_ref[:, :]
        s2r[pl.ds(b * BR, BR), :] = jax.nn.sigmoid(jnp.reshape(lb, (BR, 128)))
        return 0

    jax.lax.fori_loop(0, NB, logits_blk, 0)
    _topk_mask(s2r)

    li = jax.lax.broadcasted_iota(jnp.int32, (BK, 1), 0)
    lj = jax.lax.broadcasted_iota(jnp.int32, (BK, 128), 1)
    lsel = (li % 128 == lj).astype(_f32)

    def write_blk(b, _):
        sb = s2r[pl.ds(b * BR, BR), :]
        sbig = jnp.reshape(jnp.broadcast_to(
            jnp.reshape(sb, (BR, 1, 128)), (BR, 128, 128)), (BK, 128))
        s1 = jnp.sum(sbig * lsel, axis=1, keepdims=True)
        xb = src[pl.ds(b * BK, BK), pl.ds(scol, cin)]
        pad_ref[pl.ds(P0 + b * BK, BK), pl.ds(0, cin)] = xb * s1
        return 0

    jax.lax.fori_loop(0, NB, write_blk, 0)


def _dma(src, dst, sem):
    cp = pltpu.make_async_copy(src, dst, sem)
    cp.start()
    cp.wait()


def _body(xt, w0, g0, b0,
          wi11, bi11, w11, g11, b11,
          wi12, bi12, w12, g12, b12,
          wi13, bi13, w13, g13, b13,
          wi1d, bi1d, w1d, g1d, b1d,
          wi21, bi21, w21, g21, b21,
          wi22, bi22, w22, g22, b22,
          wi23, bi23, w23, g23, b23,
          wi2d, bi2d, w2d, g2d, b2d,
          out_ref, pad128, act, t128, idbuf, s2r, wbuf, sem):
    H1, O1 = 0, 32

    def zero128(b, _):
        pad128[pl.ds(b * 1160, 1160), :] = jnp.zeros((1160, 128), _f32)
        return 0

    jax.lax.fori_loop(0, NP // 1160, zero128, 0)

    _dma(xt, pad128.at[pl.ds(P0, N)], sem)

    _conv_bn(pad128, w0, g0, b0, 128, 32, act, H1, relu=True)

    _focal_stage(act, H1, wi11, bi11, pad128, 32, s2r)
    _conv_bn(pad128, w11, g11, b11, 32, 32, act, O1, relu=True)

    _focal_stage(act, O1, wi12, bi12, pad128, 32, s2r)
    _conv_bn(pad128, w12, g12, b12, 32, 32, act, O1, relu=False)

    _focal_stage(act, O1, wi13, bi13, pad128, 32, s2r)
    m3, v3 = _conv_pass(pad128, w13, 32, 128, t128, 0)

    _focal_stage(act, H1, wi1d, bi1d, pad128, 32, s2r)
    md, vd = _conv_pass(pad128, w1d, 32, 128, idbuf, 0)

    g13v, b13v = g13[:, :], b13[:, :]
    g1dv, b1dv = g1d[:, :], b1d[:, :]

    def addrelu1(b, _):
        o3n = _norm(t128[pl.ds(b * BK, BK), :], m3, v3, g13v, b13v)
        idn = _norm(idbuf[pl.ds(b * BK, BK), :], md, vd, g1dv, b1dv)
        t128[pl.ds(b * BK, BK), :] = jnp.maximum(o3n + idn, 0.0)
        return 0

    jax.lax.fori_loop(0, NB, addrelu1, 0)

    _focal_stage(t128, 0, wi21, bi21, pad128, 128, s2r)
    _conv_bn(pad128, w21, g21, b21, 128, 32, act, H1, relu=True)

    _focal_stage(act, H1, wi22, bi22, pad128, 32, s2r)
    _conv_bn(pad128, w22, g22, b22, 32, 32, act, O1, relu=False)

    _focal_stage(t128, 0, wi2d, bi2d, pad128, 128, s2r)
    stats_d = []
    for c in range(4):
        _dma(w2d.at[c], wbuf.at[pl.ds(0, 3456)], sem)
        stats_d.append(_conv_pass(pad128, wbuf, 128, 128, out_ref, c * 128))

    _focal_stage(act, O1, wi23, bi23, pad128, 32, s2r)
    for c in range(4):
        _dma(w23.at[c], wbuf.at[pl.ds(0, 864)], sem)
        m23, v23 = _conv_pass(pad128, wbuf, 32, 128, idbuf, 0)
        mdc, vdc = stats_d[c]
        g23c = g23[:, pl.ds(c * 128, 128)]
        b23c = b23[:, pl.ds(c * 128, 128)]
        g2dc = g2d[:, pl.ds(c * 128, 128)]
        b2dc = b2d[:, pl.ds(c * 128, 128)]

        def finale(b, _):
            p3n = _norm(idbuf[pl.ds(b * BK, BK), :], m23, v23, g23c, b23c)
            id2n = _norm(out_ref[pl.ds(b * BK, BK), pl.ds(c * 128, 128)],
                         mdc, vdc, g2dc, b2dc)
            out_ref[pl.ds(b * BK, BK), pl.ds(c * 128, 128)] = \
                jnp.maximum(p3n + id2n, 0.0)
            return 0

        jax.lax.fori_loop(0, NB, finale, 0)


def _prep_focal(p, stream):
    cin = p['w_imp'].shape[0]
    cout = p['W'].shape[-1]
    w = p['W'].reshape(27 * cin, cout).astype(_bf16)
    if stream:
        w = w.reshape(27 * cin, 4, 128).transpose(1, 0, 2)
    return (jnp.pad(p['w_imp'], (0, 128 - cin)).reshape(1, 128),
            p['b_imp'].reshape(1, 1),
            w,
            p['gamma'].reshape(1, cout),
            p['beta'].reshape(1, cout))


def kernel(x, params):
    xt = jnp.transpose(x, (0, 2, 3, 4, 1)).reshape(N, 128)
    ci = params['conv_input']
    args = [xt,
            ci['W'].reshape(27 * 128, 32).astype(_bf16),
            ci['gamma'].reshape(1, 32), ci['beta'].reshape(1, 32)]
    for blk in ('block1', 'block2'):
        for nm in ('conv1', 'conv2', 'conv3', 'downsample'):
            stream = (blk == 'block2' and nm in ('conv3', 'downsample'))
            args.extend(_prep_focal(params[blk][nm], stream))

    any_spec = pl.BlockSpec(memory_space=pltpu.MemorySpace.HBM)
    vmem_spec = pl.BlockSpec(memory_space=pltpu.MemorySpace.VMEM)
    specs = [any_spec] + [vmem_spec] * 33 + \
        [vmem_spec, vmem_spec, any_spec, vmem_spec, vmem_spec] + \
        [vmem_spec, vmem_spec, any_spec, vmem_spec, vmem_spec]

    out = pl.pallas_call(
        _body,
        out_shape=jax.ShapeDtypeStruct((N, 512), _f32),
        in_specs=specs,
        scratch_shapes=[
            pltpu.VMEM((NP, 128), _f32),
            pltpu.VMEM((N, 128), _f32),
            pltpu.VMEM((N, 128), _f32),
            pltpu.VMEM((N, 128), _f32),
            pltpu.VMEM((NR, 128), _f32),
            pltpu.VMEM((27 * 128, 128), _bf16),
            pltpu.SemaphoreType.DMA,
        ],
    )(*args)
    return jnp.transpose(out.reshape(1, D, H, W, 512), (0, 4, 1, 2, 3))
